# Initial kernel scaffold; baseline (speedup 1.0000x reference)
import jax
import jax.numpy as jnp
from jax.experimental import pallas as pl


def kernel(x, edge_index, adj_values, W):
    raise NotImplementedError("write your pallas kernel here")


# speedup vs baseline: 4.3752x; 4.3752x over previous
import functools

import jax
import jax.numpy as jnp
from jax import lax
from jax.experimental import pallas as pl
from jax.experimental.pallas import tpu as pltpu
from jax.experimental.pallas import tpu_sc as plsc

_LANES = 16
_CHUNK = 80
_ZROWS = 200


def _mm_body(x_ref, w_ref, o_ref):
    o_ref[...] = jnp.dot(x_ref[...], w_ref[...],
                         preferred_element_type=jnp.float32)


def _matmul(x, w):
    n, d_in = x.shape
    d_out = w.shape[1]
    blk = 1000
    return pl.pallas_call(
        _mm_body,
        grid=(n // blk,),
        in_specs=[
            pl.BlockSpec((blk, d_in), lambda i: (i, 0)),
            pl.BlockSpec((d_in, d_out), lambda i: (0, 0)),
        ],
        out_specs=pl.BlockSpec((blk, d_out), lambda i: (i, 0)),
        out_shape=jax.ShapeDtypeStruct((n, d_out), jnp.float32),
    )(x, w)


def _add_body(a_ref, b_ref, o_ref):
    o_ref[...] = a_ref[...] + b_ref[...]


def _combine(a, b):
    n, d = a.shape
    blk = 1000
    return pl.pallas_call(
        _add_body,
        grid=(n // blk,),
        in_specs=[
            pl.BlockSpec((blk, d), lambda i: (i, 0)),
            pl.BlockSpec((blk, d), lambda i: (i, 0)),
        ],
        out_specs=pl.BlockSpec((blk, d), lambda i: (i, 0)),
        out_shape=jax.ShapeDtypeStruct((n, d), jnp.float32),
    )(a, b)


def _make_spmm(n, d, e):
    info = plsc.get_sparse_core_info()
    n_cores, n_sub = info.num_cores, info.num_subcores
    nw = n_cores * n_sub
    per_tile = e // nw
    n_chunks = per_tile // _CHUNK
    n_zcopies = n // _ZROWS
    d_vregs = d // _LANES
    mesh = plsc.VectorSubcoreMesh(core_axis_name="c", subcore_axis_name="s")

    @functools.partial(
        pl.kernel,
        mesh=mesh,
        out_type=jax.ShapeDtypeStruct((n_cores, n, d), jnp.float32),
        scratch_types=[
            pltpu.VMEM((_CHUNK,), jnp.int32),
            pltpu.VMEM((_CHUNK,), jnp.int32),
            pltpu.VMEM((_CHUNK,), jnp.float32),
            pltpu.VMEM((_CHUNK, d), jnp.float32),
            pltpu.VMEM((_ZROWS, d), jnp.float32),
            pltpu.VMEM_SHARED((n, d), jnp.float32),
            pltpu.SemaphoreType.DMA,
        ],
    )
    def spmm(m_hbm, src_hbm, dst_hbm, val_hbm, out_hbm,
             src_v, dst_v, val_v, rows_v, stage_v, acc, sem):
        cid = lax.axis_index("c")
        sid = lax.axis_index("s")
        wid = cid * n_sub + sid

        zeros16 = jnp.zeros((_LANES,), jnp.float32)

        def zero_row(r, carry):
            for c in range(d_vregs):
                stage_v[r, pl.ds(c * _LANES, _LANES)] = zeros16
            return carry

        lax.fori_loop(0, _ZROWS, zero_row, 0)

        def zero_acc(j, carry):
            k = sid + j * n_sub

            @pl.when(k < n_zcopies)
            def _():
                pltpu.sync_copy(stage_v, acc.at[pl.ds(k * _ZROWS, _ZROWS)])

            return carry

        lax.fori_loop(0, (n_zcopies + n_sub - 1) // n_sub, zero_acc, 0)
        plsc.subcore_barrier()

        edge_base = wid * per_tile

        def chunk_body(i, carry):
            base = edge_base + i * _CHUNK
            pltpu.sync_copy(src_hbm.at[pl.ds(base, _CHUNK)], src_v)
            pltpu.sync_copy(dst_hbm.at[pl.ds(base, _CHUNK)], dst_v)
            pltpu.sync_copy(val_hbm.at[pl.ds(base, _CHUNK)], val_v)
            pltpu.async_copy(m_hbm.at[src_v], rows_v, sem).wait()

            def scale_group(g, c2):
                vv = val_v[pl.ds(g * _LANES, _LANES)]
                for j in range(_LANES):
                    av = vv[j]
                    r = g * _LANES + j
                    for c in range(d_vregs):
                        sl = rows_v[r, pl.ds(c * _LANES, _LANES)]
                        rows_v[r, pl.ds(c * _LANES, _LANES)] = sl * av
                return c2

            lax.fori_loop(0, _CHUNK // _LANES, scale_group, 0)
            pltpu.sync_copy(rows_v, acc.at[dst_v], add=True)
            return carry

        lax.fori_loop(0, n_chunks, chunk_body, 0)
        plsc.subcore_barrier()

        def readout(j, carry):
            k = sid + j * n_sub

            @pl.when(k < n_zcopies)
            def _():
                start = k * _ZROWS
                pltpu.sync_copy(acc.at[pl.ds(start, _ZROWS)], stage_v)
                pltpu.sync_copy(stage_v,
                                out_hbm.at[cid, pl.ds(start, _ZROWS)])

            return carry

        lax.fori_loop(0, (n_zcopies + n_sub - 1) // n_sub, readout, 0)

    return spmm


def kernel(x, edge_index, adj_values, W):
    n, d = x.shape[0], W.shape[1]
    e = edge_index.shape[1]
    m = _matmul(x, W)
    spmm = _make_spmm(n, d, e)
    parts = spmm(m, edge_index[0], edge_index[1], adj_values)
    return _combine(parts[0], parts[1])


# speedup vs baseline: 10.4904x; 2.3977x over previous
import functools

import jax
import jax.numpy as jnp
from jax import lax
from jax.experimental import pallas as pl
from jax.experimental.pallas import tpu as pltpu
from jax.experimental.pallas import tpu_sc as plsc

_LANES = 16
_CHUNK = 80


def _mm_body(x_ref, w_ref, o_ref):
    o_ref[...] = jnp.dot(x_ref[...], w_ref[...],
                         preferred_element_type=jnp.float32)


def _matmul(x, w):
    n, d_in = x.shape
    d_out = w.shape[1]
    blk = 1000
    return pl.pallas_call(
        _mm_body,
        grid=(n // blk,),
        in_specs=[
            pl.BlockSpec((blk, d_in), lambda i: (i, 0)),
            pl.BlockSpec((d_in, d_out), lambda i: (0, 0)),
        ],
        out_specs=pl.BlockSpec((blk, d_out), lambda i: (i, 0)),
        out_shape=jax.ShapeDtypeStruct((n, d_out), jnp.float32),
    )(x, w)


def _add_body(a_ref, b_ref, o_ref):
    o_ref[...] = a_ref[...] + b_ref[...]


def _combine(a, b):
    n, d = a.shape
    blk = 1000
    return pl.pallas_call(
        _add_body,
        grid=(n // blk,),
        in_specs=[
            pl.BlockSpec((blk, d), lambda i: (i, 0)),
            pl.BlockSpec((blk, d), lambda i: (i, 0)),
        ],
        out_specs=pl.BlockSpec((blk, d), lambda i: (i, 0)),
        out_shape=jax.ShapeDtypeStruct((n, d), jnp.float32),
    )(a, b)


def _make_spmm(n, d, e):
    info = plsc.get_sparse_core_info()
    n_cores, n_sub = info.num_cores, info.num_subcores
    nw = n_cores * n_sub
    per_tile = e // nw
    n_chunks = per_tile // _CHUNK
    n_pairs = (n_chunks - 1) // 2
    n_zcopies = n // _CHUNK
    d_vregs = d // _LANES
    groups = _CHUNK // _LANES
    mesh = plsc.VectorSubcoreMesh(core_axis_name="c", subcore_axis_name="s")

    @functools.partial(
        pl.kernel,
        mesh=mesh,
        out_type=jax.ShapeDtypeStruct((n_cores, n, d), jnp.float32),
        scratch_types=[
            pltpu.VMEM((per_tile,), jnp.int32),
            pltpu.VMEM((per_tile,), jnp.float32),
            pltpu.VMEM((_CHUNK,), jnp.int32),
            pltpu.VMEM((_CHUNK,), jnp.int32),
            pltpu.VMEM((_CHUNK, d), jnp.float32),
            pltpu.VMEM((_CHUNK, d), jnp.float32),
            pltpu.VMEM_SHARED((n, d), jnp.float32),
            pltpu.SemaphoreType.DMA,
            pltpu.SemaphoreType.DMA,
            pltpu.SemaphoreType.DMA,
        ],
    )
    def spmm(m_hbm, src_hbm, dst_hbm, val_hbm, out_hbm,
             src_v, val_v, dstb0, dstb1, rows0, rows1, acc,
             isem, gsem0, gsem1):
        cid = lax.axis_index("c")
        sid = lax.axis_index("s")
        wid = cid * n_sub + sid
        ebase = wid * per_tile

        c_src = pltpu.async_copy(
            src_hbm.at[pl.ds(ebase, per_tile)], src_v, isem)
        c_val = pltpu.async_copy(
            val_hbm.at[pl.ds(ebase, per_tile)], val_v, isem)

        zeros16 = jnp.zeros((_LANES,), jnp.float32)

        def zero_row(r, carry):
            for c in range(d_vregs):
                rows0[r, pl.ds(c * _LANES, _LANES)] = zeros16
            return carry

        lax.fori_loop(0, _CHUNK, zero_row, 0)

        def zero_acc(j, carry):
            k = sid + j * n_sub

            @pl.when(k < n_zcopies)
            def _():
                pltpu.sync_copy(rows0, acc.at[pl.ds(k * _CHUNK, _CHUNK)])

            return carry

        lax.fori_loop(0, (n_zcopies + n_sub - 1) // n_sub, zero_acc, 0)
        c_src.wait()
        c_val.wait()
        plsc.subcore_barrier()

        def scale(buf, c):
            def scale_group(g, c2):
                vv = val_v[pl.ds(c * _CHUNK + g * _LANES, _LANES)]
                for j in range(_LANES):
                    av = vv[j]
                    r = g * _LANES + j
                    for k in range(d_vregs):
                        sl = buf[r, pl.ds(k * _LANES, _LANES)]
                        buf[r, pl.ds(k * _LANES, _LANES)] = sl * av
                return c2

            lax.fori_loop(0, groups, scale_group, 0)

        def fetch(c, rbuf, dbuf, sem):
            pltpu.async_copy(dst_hbm.at[pl.ds(ebase + c * _CHUNK, _CHUNK)],
                             dbuf, sem)
            pltpu.async_copy(m_hbm.at[src_v.at[pl.ds(c * _CHUNK, _CHUNK)]],
                             rbuf, sem)

        def drain(c, rbuf, dbuf, sem):
            pltpu.make_async_copy(
                dst_hbm.at[pl.ds(ebase + c * _CHUNK, _CHUNK)], dbuf,
                sem).wait()
            pltpu.make_async_copy(
                m_hbm.at[src_v.at[pl.ds(c * _CHUNK, _CHUNK)]], rbuf,
                sem).wait()

        def scatter(rbuf, dbuf):
            pltpu.sync_copy(rbuf, acc.at[dbuf], add=True)

        fetch(0, rows0, dstb0, gsem0)

        def pair_body(i, carry):
            a = 2 * i
            fetch(a + 1, rows1, dstb1, gsem1)
            drain(a, rows0, dstb0, gsem0)
            scale(rows0, a)
            scatter(rows0, dstb0)
            fetch(a + 2, rows0, dstb0, gsem0)
            drain(a + 1, rows1, dstb1, gsem1)
            scale(rows1, a + 1)
            scatter(rows1, dstb1)
            return carry

        lax.fori_loop(0, n_pairs, pair_body, 0)
        last = n_chunks - 1
        drain(last, rows0, dstb0, gsem0)
        scale(rows0, last)
        scatter(rows0, dstb0)
        plsc.subcore_barrier()

        def readout(j, carry):
            k = sid + j * n_sub

            @pl.when(k < n_zcopies)
            def _():
                start = k * _CHUNK
                pltpu.sync_copy(acc.at[pl.ds(start, _CHUNK)], rows0)
                pltpu.sync_copy(rows0, out_hbm.at[cid, pl.ds(start, _CHUNK)])

            return carry

        lax.fori_loop(0, (n_zcopies + n_sub - 1) // n_sub, readout, 0)

    return spmm


def kernel(x, edge_index, adj_values, W):
    n, d = x.shape[0], W.shape[1]
    e = edge_index.shape[1]
    m = _matmul(x, W)
    spmm = _make_spmm(n, d, e)
    parts = spmm(m, edge_index[0], edge_index[1], adj_values)
    return _combine(parts[0], parts[1])


# speedup vs baseline: 11.5981x; 1.1056x over previous
import functools

import jax
import jax.numpy as jnp
from jax import lax
from jax.experimental import pallas as pl
from jax.experimental.pallas import tpu as pltpu
from jax.experimental.pallas import tpu_sc as plsc

_LANES = 16
_CHUNK = 80


def _mm_body(x_ref, w_ref, o_ref):
    o_ref[...] = jnp.dot(x_ref[...], w_ref[...],
                         preferred_element_type=jnp.float32)


def _matmul(x, w):
    n, d_in = x.shape
    d_out = w.shape[1]
    blk = 1000
    return pl.pallas_call(
        _mm_body,
        grid=(n // blk,),
        in_specs=[
            pl.BlockSpec((blk, d_in), lambda i: (i, 0)),
            pl.BlockSpec((d_in, d_out), lambda i: (0, 0)),
        ],
        out_specs=pl.BlockSpec((blk, d_out), lambda i: (i, 0)),
        out_shape=jax.ShapeDtypeStruct((n, d_out), jnp.float32),
    )(x, w)


def _add_body(a_ref, b_ref, o_ref):
    o_ref[...] = a_ref[...] + b_ref[...]


def _combine(a, b):
    n, d = a.shape
    blk = 1000
    return pl.pallas_call(
        _add_body,
        grid=(n // blk,),
        in_specs=[
            pl.BlockSpec((blk, d), lambda i: (i, 0)),
            pl.BlockSpec((blk, d), lambda i: (i, 0)),
        ],
        out_specs=pl.BlockSpec((blk, d), lambda i: (i, 0)),
        out_shape=jax.ShapeDtypeStruct((n, d), jnp.float32),
    )(a, b)


def _make_spmm(n, d, e):
    info = plsc.get_sparse_core_info()
    n_cores, n_sub = info.num_cores, info.num_subcores
    nw = n_cores * n_sub
    per_tile = e // nw
    n_chunks = per_tile // _CHUNK
    n_trips = (n_chunks - 2) // 3
    n_zcopies = n // _CHUNK
    d_vregs = d // _LANES
    groups = _CHUNK // _LANES
    mesh = plsc.VectorSubcoreMesh(core_axis_name="c", subcore_axis_name="s")

    @functools.partial(
        pl.kernel,
        mesh=mesh,
        out_type=jax.ShapeDtypeStruct((n_cores, n, d), jnp.float32),
        scratch_types=[
            pltpu.VMEM((per_tile,), jnp.int32),
            pltpu.VMEM((_CHUNK,), jnp.float32),
            pltpu.VMEM((_CHUNK,), jnp.float32),
            pltpu.VMEM((_CHUNK,), jnp.float32),
            pltpu.VMEM((_CHUNK,), jnp.int32),
            pltpu.VMEM((_CHUNK,), jnp.int32),
            pltpu.VMEM((_CHUNK,), jnp.int32),
            pltpu.VMEM((_CHUNK, d), jnp.float32),
            pltpu.VMEM((_CHUNK, d), jnp.float32),
            pltpu.VMEM((_CHUNK, d), jnp.float32),
            pltpu.VMEM_SHARED((n, d), jnp.float32),
            pltpu.SemaphoreType.DMA,
            pltpu.SemaphoreType.DMA,
            pltpu.SemaphoreType.DMA,
            pltpu.SemaphoreType.DMA,
            pltpu.SemaphoreType.DMA,
            pltpu.SemaphoreType.DMA,
            pltpu.SemaphoreType.DMA,
        ],
    )
    def spmm(m_hbm, src_hbm, dst_hbm, val_hbm, out_hbm,
             src_v, valb0, valb1, valb2, dstb0, dstb1, dstb2,
             rows0, rows1, rows2, acc,
             isem, gsem0, gsem1, gsem2, ssem0, ssem1, ssem2):
        cid = lax.axis_index("c")
        sid = lax.axis_index("s")
        wid = cid * n_sub + sid
        ebase = wid * per_tile

        c_src = pltpu.async_copy(
            src_hbm.at[pl.ds(ebase, per_tile)], src_v, isem)

        zeros16 = jnp.zeros((_LANES,), jnp.float32)

        def zero_row(r, carry):
            for c in range(d_vregs):
                rows0[r, pl.ds(c * _LANES, _LANES)] = zeros16
            return carry

        lax.fori_loop(0, _CHUNK, zero_row, 0)

        def zero_acc(j, carry):
            k = sid + j * n_sub

            @pl.when(k < n_zcopies)
            def _():
                pltpu.sync_copy(rows0, acc.at[pl.ds(k * _CHUNK, _CHUNK)])

            return carry

        lax.fori_loop(0, (n_zcopies + n_sub - 1) // n_sub, zero_acc, 0)
        c_src.wait()

        rbufs = (rows0, rows1, rows2)
        vbufs = (valb0, valb1, valb2)
        dbufs = (dstb0, dstb1, dstb2)
        gsems = (gsem0, gsem1, gsem2)
        ssems = (ssem0, ssem1, ssem2)

        def scale(buf, vbuf):
            def scale_group(g, c2):
                vv = vbuf[pl.ds(g * _LANES, _LANES)]
                for j in range(_LANES):
                    av = vv[j]
                    r = g * _LANES + j
                    for k in range(d_vregs):
                        sl = buf[r, pl.ds(k * _LANES, _LANES)]
                        buf[r, pl.ds(k * _LANES, _LANES)] = sl * av
                return c2

            lax.fori_loop(0, groups, scale_group, 0)

        def fetch(c, b):
            pltpu.async_copy(dst_hbm.at[pl.ds(ebase + c * _CHUNK, _CHUNK)],
                             dbufs[b], gsems[b])
            pltpu.async_copy(val_hbm.at[pl.ds(ebase + c * _CHUNK, _CHUNK)],
                             vbufs[b], gsems[b])
            pltpu.async_copy(m_hbm.at[src_v.at[pl.ds(c * _CHUNK, _CHUNK)]],
                             rbufs[b], gsems[b])

        def drain(c, b):
            pltpu.make_async_copy(
                dst_hbm.at[pl.ds(ebase + c * _CHUNK, _CHUNK)], dbufs[b],
                gsems[b]).wait()
            pltpu.make_async_copy(
                val_hbm.at[pl.ds(ebase + c * _CHUNK, _CHUNK)], vbufs[b],
                gsems[b]).wait()
            pltpu.make_async_copy(
                m_hbm.at[src_v.at[pl.ds(c * _CHUNK, _CHUNK)]], rbufs[b],
                gsems[b]).wait()

        def wait_scatter(b):
            pltpu.make_async_copy(rbufs[b], acc.at[dbufs[b]],
                                  ssems[b]).wait()

        def process(c, b, wait_prev, guard_fetch):
            drain(c, b)
            scale(rbufs[b], vbufs[b])
            pltpu.async_copy(rbufs[b], acc.at[dbufs[b]], ssems[b],
                             add=True)
            nb = (b + 2) % 3
            if wait_prev:
                wait_scatter(nb)
            if guard_fetch:
                @pl.when(c + 2 < n_chunks)
                def _():
                    fetch(c + 2, nb)
            else:
                fetch(c + 2, nb)

        fetch(0, 0)
        fetch(1, 1)
        plsc.subcore_barrier()
        process(0, 0, wait_prev=False, guard_fetch=False)
        process(1, 1, wait_prev=True, guard_fetch=False)

        def trip_body(i, carry):
            c = 3 * i + 2
            process(c, 2, wait_prev=True, guard_fetch=True)
            process(c + 1, 0, wait_prev=True, guard_fetch=True)
            process(c + 2, 1, wait_prev=True, guard_fetch=True)
            return carry

        lax.fori_loop(0, n_trips, trip_body, 0)
        wait_scatter((n_chunks - 1) % 3)
        plsc.subcore_barrier()

        def readout(j, carry):
            k = sid + j * n_sub

            @pl.when(k < n_zcopies)
            def _():
                start = k * _CHUNK
                pltpu.sync_copy(acc.at[pl.ds(start, _CHUNK)], rows0)
                pltpu.sync_copy(rows0, out_hbm.at[cid, pl.ds(start, _CHUNK)])

            return carry

        lax.fori_loop(0, (n_zcopies + n_sub - 1) // n_sub, readout, 0)

    return spmm


def kernel(x, edge_index, adj_values, W):
    n, d = x.shape[0], W.shape[1]
    e = edge_index.shape[1]
    m = _matmul(x, W)
    spmm = _make_spmm(n, d, e)
    parts = spmm(m, edge_index[0], edge_index[1], adj_values)
    return _combine(parts[0], parts[1])


# speedup vs baseline: 12.1570x; 1.0482x over previous
import functools

import jax
import jax.numpy as jnp
from jax import lax
from jax.experimental import pallas as pl
from jax.experimental.pallas import tpu as pltpu
from jax.experimental.pallas import tpu_sc as plsc

_LANES = 16
_CHUNK = 80


def _addmm_body(a_ref, b_ref, w_ref, o_ref):
    o_ref[...] = jnp.dot(a_ref[...] + b_ref[...], w_ref[...],
                         preferred_element_type=jnp.float32)


def _combine_matmul(a, b, w):
    n, d_in = a.shape
    d_out = w.shape[1]
    blk = 1000
    return pl.pallas_call(
        _addmm_body,
        grid=(n // blk,),
        in_specs=[
            pl.BlockSpec((blk, d_in), lambda i: (i, 0)),
            pl.BlockSpec((blk, d_in), lambda i: (i, 0)),
            pl.BlockSpec((d_in, d_out), lambda i: (0, 0)),
        ],
        out_specs=pl.BlockSpec((blk, d_out), lambda i: (i, 0)),
        out_shape=jax.ShapeDtypeStruct((n, d_out), jnp.float32),
    )(a, b, w)


def _make_spmm(n, d, e):
    info = plsc.get_sparse_core_info()
    n_cores, n_sub = info.num_cores, info.num_subcores
    nw = n_cores * n_sub
    per_tile = e // nw
    n_chunks = per_tile // _CHUNK
    n_trips = (n_chunks - 2) // 3
    n_zcopies = n // _CHUNK
    d_vregs = d // _LANES
    groups = _CHUNK // _LANES
    mesh = plsc.VectorSubcoreMesh(core_axis_name="c", subcore_axis_name="s")

    @functools.partial(
        pl.kernel,
        mesh=mesh,
        out_type=jax.ShapeDtypeStruct((n_cores, n, d), jnp.float32),
        scratch_types=[
            pltpu.VMEM((per_tile,), jnp.int32),
            pltpu.VMEM((_CHUNK,), jnp.float32),
            pltpu.VMEM((_CHUNK,), jnp.float32),
            pltpu.VMEM((_CHUNK,), jnp.float32),
            pltpu.VMEM((_CHUNK,), jnp.int32),
            pltpu.VMEM((_CHUNK,), jnp.int32),
            pltpu.VMEM((_CHUNK,), jnp.int32),
            pltpu.VMEM((_CHUNK, d), jnp.float32),
            pltpu.VMEM((_CHUNK, d), jnp.float32),
            pltpu.VMEM((_CHUNK, d), jnp.float32),
            pltpu.VMEM_SHARED((n, d), jnp.float32),
            pltpu.SemaphoreType.DMA,
            pltpu.SemaphoreType.DMA,
            pltpu.SemaphoreType.DMA,
            pltpu.SemaphoreType.DMA,
            pltpu.SemaphoreType.DMA,
            pltpu.SemaphoreType.DMA,
            pltpu.SemaphoreType.DMA,
        ],
    )
    def spmm(m_hbm, src_hbm, dst_hbm, val_hbm, out_hbm,
             src_v, valb0, valb1, valb2, dstb0, dstb1, dstb2,
             rows0, rows1, rows2, acc,
             isem, gsem0, gsem1, gsem2, ssem0, ssem1, ssem2):
        cid = lax.axis_index("c")
        sid = lax.axis_index("s")
        wid = cid * n_sub + sid
        ebase = wid * per_tile

        c_src = pltpu.async_copy(
            src_hbm.at[pl.ds(ebase, per_tile)], src_v, isem)

        zeros16 = jnp.zeros((_LANES,), jnp.float32)

        def zero_row(r, carry):
            for c in range(d_vregs):
                rows0[r, pl.ds(c * _LANES, _LANES)] = zeros16
            return carry

        lax.fori_loop(0, _CHUNK, zero_row, 0)

        def zero_acc(j, carry):
            k = sid + j * n_sub

            @pl.when(k < n_zcopies)
            def _():
                pltpu.sync_copy(rows0, acc.at[pl.ds(k * _CHUNK, _CHUNK)])

            return carry

        lax.fori_loop(0, (n_zcopies + n_sub - 1) // n_sub, zero_acc, 0)
        c_src.wait()

        rbufs = (rows0, rows1, rows2)
        vbufs = (valb0, valb1, valb2)
        dbufs = (dstb0, dstb1, dstb2)
        gsems = (gsem0, gsem1, gsem2)
        ssems = (ssem0, ssem1, ssem2)

        def scale(buf, vbuf):
            def scale_group(g, c2):
                vv = vbuf[pl.ds(g * _LANES, _LANES)]
                for j in range(_LANES):
                    av = vv[j]
                    r = g * _LANES + j
                    for k in range(d_vregs):
                        sl = buf[r, pl.ds(k * _LANES, _LANES)]
                        buf[r, pl.ds(k * _LANES, _LANES)] = sl * av
                return c2

            lax.fori_loop(0, groups, scale_group, 0)

        def fetch(c, b):
            pltpu.async_copy(dst_hbm.at[pl.ds(ebase + c * _CHUNK, _CHUNK)],
                             dbufs[b], gsems[b])
            pltpu.async_copy(val_hbm.at[pl.ds(ebase + c * _CHUNK, _CHUNK)],
                             vbufs[b], gsems[b])
            pltpu.async_copy(m_hbm.at[src_v.at[pl.ds(c * _CHUNK, _CHUNK)]],
                             rbufs[b], gsems[b])

        def drain(c, b):
            pltpu.make_async_copy(
                dst_hbm.at[pl.ds(ebase + c * _CHUNK, _CHUNK)], dbufs[b],
                gsems[b]).wait()
            pltpu.make_async_copy(
                val_hbm.at[pl.ds(ebase + c * _CHUNK, _CHUNK)], vbufs[b],
                gsems[b]).wait()
            pltpu.make_async_copy(
                m_hbm.at[src_v.at[pl.ds(c * _CHUNK, _CHUNK)]], rbufs[b],
                gsems[b]).wait()

        def wait_scatter(b):
            pltpu.make_async_copy(rbufs[b], acc.at[dbufs[b]],
                                  ssems[b]).wait()

        def process(c, b, wait_prev, guard_fetch):
            drain(c, b)
            scale(rbufs[b], vbufs[b])
            pltpu.async_copy(rbufs[b], acc.at[dbufs[b]], ssems[b],
                             add=True)
            nb = (b + 2) % 3
            if wait_prev:
                wait_scatter(nb)
            if guard_fetch:
                @pl.when(c + 2 < n_chunks)
                def _():
                    fetch(c + 2, nb)
            else:
                fetch(c + 2, nb)

        fetch(0, 0)
        fetch(1, 1)
        plsc.subcore_barrier()
        process(0, 0, wait_prev=False, guard_fetch=False)
        process(1, 1, wait_prev=True, guard_fetch=False)

        def trip_body(i, carry):
            c = 3 * i + 2
            process(c, 2, wait_prev=True, guard_fetch=True)
            process(c + 1, 0, wait_prev=True, guard_fetch=True)
            process(c + 2, 1, wait_prev=True, guard_fetch=True)
            return carry

        lax.fori_loop(0, n_trips, trip_body, 0)
        wait_scatter((n_chunks - 1) % 3)
        plsc.subcore_barrier()

        def readout(j, carry):
            k = sid + j * n_sub

            @pl.when(k < n_zcopies)
            def _():
                start = k * _CHUNK
                pltpu.sync_copy(acc.at[pl.ds(start, _CHUNK)], rows0)
                pltpu.sync_copy(rows0, out_hbm.at[cid, pl.ds(start, _CHUNK)])

            return carry

        lax.fori_loop(0, (n_zcopies + n_sub - 1) // n_sub, readout, 0)

    return spmm


def kernel(x, edge_index, adj_values, W):
    n, d = x.shape
    e = edge_index.shape[1]
    spmm = _make_spmm(n, d, e)
    parts = spmm(x, edge_index[0], edge_index[1], adj_values)
    return _combine_matmul(parts[0], parts[1], W)
